# baseline (speedup 1.0000x reference)
import dataclasses

import jax
import jax.numpy as jnp
from jax.experimental import pallas as pl
from jax.experimental.pallas import tpu as pltpu
from jax.experimental.pallas import tpu_sc as plsc

_THRESHOLD = 0.3
_PENALTY_WEIGHT = 0.05

_B, _H, _W, _C = 8, 384, 384, 96
_BH = 32
_NH = _H // _BH
_NSTEPS = _B * _NH


def _stream(xp_ref, xt_ref, rowp, rowt, colp, colt, zp, zt):
    b = pl.program_id(0)
    h = pl.program_id(1)
    i = b * _NH + h
    xp = xp_ref[0]
    xt = xt_ref[0]

    rowp[i, :] = jnp.max(jnp.max(xp, axis=1), axis=1)
    rowt[i, :] = jnp.max(jnp.max(xt, axis=1), axis=1)
    zp_part = jnp.max(xp, axis=0)
    zt_part = jnp.max(xt, axis=0)

    @pl.when(h == 0)
    def _():
        zp[...] = zp_part
        zt[...] = zt_part

    @pl.when(h != 0)
    def _():
        zp[...] = jnp.maximum(zp[...], zp_part)
        zt[...] = jnp.maximum(zt[...], zt_part)

    @pl.when(h == _NH - 1)
    def _():
        colp[b, :] = jnp.max(zp[...], axis=0)
        colt[b, :] = jnp.max(zt[...], axis=0)


def _sc_compiler_params():
    cp = pltpu.CompilerParams()
    if "needs_layout_passes" in pltpu.CompilerParams.__dataclass_fields__:
        cp = dataclasses.replace(cp, needs_layout_passes=False)
    return cp


def _sc_sqrt(x):
    xc = jnp.maximum(x, jnp.float32(1e-20))
    i = jax.lax.bitcast_convert_type(xc, jnp.int32)
    i = jnp.int32(0x5F3759DF) - jax.lax.shift_right_logical(i, 1)
    y = jax.lax.bitcast_convert_type(i, jnp.float32)
    for _ in range(3):
        y = y * (jnp.float32(1.5) - jnp.float32(0.5) * xc * y * y)
    return x * y


def _sc_bounds(chunks, thr, size):
    lane = jax.lax.broadcasted_iota(jnp.int32, (16,), 0).astype(jnp.float32)
    size_f = jnp.float32(size)
    mn = jnp.full((16,), size_f)
    mx = jnp.full((16,), jnp.float32(-1.0))
    for v, base in chunks:
        m = v > thr
        idx = lane + jnp.float32(base)
        mn = jnp.minimum(mn, jnp.where(m, idx, size_f))
        mx = jnp.maximum(mx, jnp.where(m, idx, jnp.float32(-1.0)))
    mns = jnp.full((16,), jnp.min(mn))
    mxs = jnp.full((16,), jnp.max(mx))
    empty = mns == size_f
    return (jnp.where(empty, jnp.float32(0.0), mns),
            jnp.where(empty, jnp.float32(1.0), mxs))


def _sc_epilogue(rowp_hbm, rowt_hbm, colp_hbm, colt_hbm, o_hbm,
                 vrp, vrt, vcp, vct, vout, sem):
    c = jax.lax.axis_index("c")
    s = jax.lax.axis_index("s")

    @pl.when(jnp.logical_and(c == 0, s == 0))
    def _():
        pltpu.async_copy(rowp_hbm, vrp, sem).wait()
        pltpu.async_copy(rowt_hbm, vrt, sem).wait()
        pltpu.async_copy(colp_hbm, vcp, sem).wait()
        pltpu.async_copy(colt_hbm, vct, sem).wait()

        def row_chunks(ref, bb):
            return [(ref[bb * _NH + j // 2, pl.ds((j % 2) * 16, 16)], 16.0 * j)
                    for j in range(2 * _NH)]

        def col_chunks(ref, bb):
            return [(ref[bb, pl.ds(k * 16, 16)], 16.0 * k)
                    for k in range(_W // 16)]

        total = jnp.zeros((16,), jnp.float32)
        for bb in range(_B):
            p_y1, p_y2 = _sc_bounds(row_chunks(vrp, bb), _THRESHOLD, _H)
            p_x1, p_x2 = _sc_bounds(col_chunks(vcp, bb), _THRESHOLD, _W)
            t_y1, t_y2 = _sc_bounds(row_chunks(vrt, bb), 0.5, _H)
            t_x1, t_x2 = _sc_bounds(col_chunks(vct, bb), 0.5, _W)

            pred_area = (p_y2 - p_y1 + 1.0) * (p_x2 - p_x1 + 1.0)
            true_area = (t_y2 - t_y1 + 1.0) * (t_x2 - t_x1 + 1.0)
            area_penalty = (jnp.maximum(pred_area - true_area, 0.0)
                            / (true_area + 1.0))
            dy = (p_y1 + p_y2 - t_y1 - t_y2) * 0.5
            dx = (p_x1 + p_x2 - t_x1 - t_x2) * 0.5
            center_offset = _sc_sqrt(dy * dy + dx * dx) / 20.0
            total = total + area_penalty + center_offset

        vout[...] = (_PENALTY_WEIGHT / _B) * total
        pltpu.async_copy(vout, o_hbm, sem).wait()


def kernel(prediction_probs, expected_onehot):
    xp = jnp.transpose(prediction_probs, (0, 1, 3, 2))
    xt = jnp.transpose(expected_onehot, (0, 1, 3, 2))
    rowp, rowt, colp, colt = pl.pallas_call(
        _stream,
        grid=(_B, _NH),
        in_specs=[
            pl.BlockSpec((1, _BH, _C, _W), lambda b, h: (b, h, 0, 0)),
            pl.BlockSpec((1, _BH, _C, _W), lambda b, h: (b, h, 0, 0)),
        ],
        out_specs=[
            pl.BlockSpec((_NSTEPS, _BH), lambda b, h: (0, 0)),
            pl.BlockSpec((_NSTEPS, _BH), lambda b, h: (0, 0)),
            pl.BlockSpec((_B, _W), lambda b, h: (0, 0)),
            pl.BlockSpec((_B, _W), lambda b, h: (0, 0)),
        ],
        out_shape=[
            jax.ShapeDtypeStruct((_NSTEPS, _BH), jnp.float32),
            jax.ShapeDtypeStruct((_NSTEPS, _BH), jnp.float32),
            jax.ShapeDtypeStruct((_B, _W), jnp.float32),
            jax.ShapeDtypeStruct((_B, _W), jnp.float32),
        ],
        scratch_shapes=[
            pltpu.VMEM((_C, _W), jnp.float32),
            pltpu.VMEM((_C, _W), jnp.float32),
        ],
    )(xp, xt)

    sc_kernel = pl.kernel(
        _sc_epilogue,
        out_type=jax.ShapeDtypeStruct((16,), jnp.float32),
        mesh=plsc.VectorSubcoreMesh(core_axis_name="c", subcore_axis_name="s"),
        scratch_types=[
            pltpu.VMEM((_NSTEPS, _BH), jnp.float32),
            pltpu.VMEM((_NSTEPS, _BH), jnp.float32),
            pltpu.VMEM((_B, _W), jnp.float32),
            pltpu.VMEM((_B, _W), jnp.float32),
            pltpu.VMEM((16,), jnp.float32),
            pltpu.SemaphoreType.DMA,
        ],
        compiler_params=_sc_compiler_params(),
    )
    out = sc_kernel(rowp, rowt, colp, colt)
    return out[0]
